# baseline (speedup 1.0000x reference)
import functools

import jax
import jax.numpy as jnp
from jax import lax
from jax.experimental import pallas as pl


def _ce_body(tgt_ref, x_ref, out_ref, *, num_tokens, nblocks, vocab, chunk):
    nchunks = vocab // chunk
    tb = x_ref.shape[0]

    lane = jax.lax.broadcasted_iota(jnp.int32, (tb, chunk), 1)
    t_b = tgt_ref[...]

    def body_a(c, carry):
        m, tgt = carry
        x = x_ref[:, pl.ds(c * chunk, chunk)]
        eq = (lane + c * chunk) == t_b
        return jnp.maximum(m, x), tgt + jnp.where(eq, x, 0.0)

    m0 = jnp.full((tb, chunk), -jnp.inf, jnp.float32)
    t0 = jnp.zeros((tb, chunk), jnp.float32)
    m_l, tgt_l = lax.fori_loop(0, nchunks, body_a, (m0, t0), unroll=False)

    m_row = jnp.max(m_l, axis=-1, keepdims=True)
    tgt_row = jnp.sum(tgt_l, axis=-1, keepdims=True)

    def body_b(c, s):
        x = x_ref[:, pl.ds(c * chunk, chunk)]
        return s + jnp.exp(x - m_row)

    s_l = lax.fori_loop(
        0, nchunks, body_b, jnp.zeros((tb, chunk), jnp.float32), unroll=False
    )
    s_row = jnp.sum(s_l, axis=-1, keepdims=True)

    lse = m_row + jnp.log(s_row)
    part = jnp.sum(lse - tgt_row, keepdims=True).reshape(1, 1)

    i = pl.program_id(0)

    @pl.when(i == 0)
    def _init():
        out_ref[...] = jnp.zeros((1, 1), jnp.float32)

    out_ref[...] += part

    @pl.when(i == nblocks - 1)
    def _fin():
        out_ref[...] = out_ref[...] * (1.0 / num_tokens)


@functools.partial(jax.jit, static_argnames=("block_tokens", "chunk"))
def _ce_mean(logits_flat, targets_col, block_tokens, chunk):
    num_tokens, vocab = logits_flat.shape
    nblocks = num_tokens // block_tokens
    body = functools.partial(
        _ce_body, num_tokens=num_tokens, nblocks=nblocks, vocab=vocab,
        chunk=chunk,
    )
    out = pl.pallas_call(
        body,
        grid=(nblocks,),
        in_specs=[
            pl.BlockSpec((block_tokens, 1), lambda i: (i, 0)),
            pl.BlockSpec((block_tokens, vocab), lambda i: (i, 0)),
        ],
        out_specs=pl.BlockSpec((1, 1), lambda i: (0, 0)),
        out_shape=jax.ShapeDtypeStruct((1, 1), jnp.float32),
    )(targets_col, logits_flat)
    return out[0, 0]


def kernel(logits, targets):
    vocab = logits.shape[-1]
    logits_flat = logits.reshape(-1, vocab)
    targets_col = targets.reshape(-1, 1).astype(jnp.int32)
    return _ce_mean(logits_flat, targets_col, 128, 128)


# speedup vs baseline: 4.9005x; 4.9005x over previous
import functools

import jax
import jax.numpy as jnp
from jax import lax
from jax.experimental import pallas as pl


def _ce_body(tgt_ref, x_ref, out_ref, *, num_tokens, nblocks, vocab, chunk):
    nchunks = vocab // chunk
    tb = x_ref.shape[0]

    lane = jax.lax.broadcasted_iota(jnp.int32, (tb, chunk), 1)
    t_b = tgt_ref[...]

    m_l = jnp.full((tb, chunk), -jnp.inf, jnp.float32)
    tgt_l = jnp.zeros((tb, chunk), jnp.float32)
    for c in range(nchunks):
        x = x_ref[:, c * chunk:(c + 1) * chunk]
        eq = (lane + c * chunk) == t_b
        m_l = jnp.maximum(m_l, x)
        tgt_l = tgt_l + jnp.where(eq, x, 0.0)

    m_row = jnp.max(m_l, axis=-1, keepdims=True)
    tgt_row = jnp.sum(tgt_l, axis=-1, keepdims=True)

    s_l = jnp.zeros((tb, chunk), jnp.float32)
    for c in range(nchunks):
        x = x_ref[:, c * chunk:(c + 1) * chunk]
        s_l = s_l + jnp.exp(x - m_row)
    s_row = jnp.sum(s_l, axis=-1, keepdims=True)

    lse = m_row + jnp.log(s_row)
    part = jnp.sum(lse - tgt_row, keepdims=True).reshape(1, 1)

    i = pl.program_id(0)

    @pl.when(i == 0)
    def _init():
        out_ref[...] = jnp.zeros((1, 1), jnp.float32)

    out_ref[...] += part

    @pl.when(i == nblocks - 1)
    def _fin():
        out_ref[...] = out_ref[...] * (1.0 / num_tokens)


@functools.partial(jax.jit, static_argnames=("block_tokens", "chunk"))
def _ce_mean(logits_flat, targets_col, block_tokens, chunk):
    num_tokens, vocab = logits_flat.shape
    nblocks = num_tokens // block_tokens
    body = functools.partial(
        _ce_body, num_tokens=num_tokens, nblocks=nblocks, vocab=vocab,
        chunk=chunk,
    )
    out = pl.pallas_call(
        body,
        grid=(nblocks,),
        in_specs=[
            pl.BlockSpec((block_tokens, 1), lambda i: (i, 0)),
            pl.BlockSpec((block_tokens, vocab), lambda i: (i, 0)),
        ],
        out_specs=pl.BlockSpec((1, 1), lambda i: (0, 0)),
        out_shape=jax.ShapeDtypeStruct((1, 1), jnp.float32),
    )(targets_col, logits_flat)
    return out[0, 0]


def kernel(logits, targets):
    vocab = logits.shape[-1]
    logits_flat = logits.reshape(-1, vocab)
    targets_col = targets.reshape(-1, 1).astype(jnp.int32)
    return _ce_mean(logits_flat, targets_col, 128, 128)
